# baseline (speedup 1.0000x reference)
import functools

import jax
import jax.numpy as jnp
from jax import lax
from jax.experimental import pallas as pl
from jax.experimental.pallas import tpu as pltpu
from jax.experimental.pallas import tpu_sc as plsc

BATCH = 4096
HIST = 200
D = 64
B = BATCH * HIST
NC = 2
NS = 16
NW = NC * NS
R = B // NW
CH = 128
NCH = R // CH
K = 5
NG = NCH // K

_mesh = plsc.VectorSubcoreMesh(core_axis_name="c", subcore_axis_name="s")


@functools.partial(
    pl.kernel,
    mesh=_mesh,
    out_type=jax.ShapeDtypeStruct((NW, NG, K, CH, D), jnp.float32),
    scratch_types=[
        pltpu.VMEM((NCH, CH), jnp.int32),
        pltpu.VMEM((K, CH, D), jnp.float32),
        pltpu.SemaphoreType.DMA,
    ],
    compiler_params=pltpu.CompilerParams(use_tc_tiling_on_sc=False),
)
def _emb_lookup(idx_hbm, table_hbm, out_hbm, idx_v, rows_v, gsem):
    cid = lax.axis_index("c")
    sid = lax.axis_index("s")
    wid = sid * NC + cid
    pltpu.sync_copy(idx_hbm.at[wid], idx_v)

    def group(g, _):
        for j in range(K):
            pltpu.async_copy(
                table_hbm.at[idx_v.at[g * K + j]], rows_v.at[j], gsem
            )
        for j in range(K):
            pltpu.make_async_copy(
                table_hbm.at[idx_v.at[0]], rows_v.at[j], gsem
            ).wait()
        pltpu.sync_copy(rows_v, out_hbm.at[wid, g])
        return 0

    lax.fori_loop(0, NG, group, 0)


def kernel(items, table):
    idx = items.astype(jnp.int32).reshape(NW, NCH, CH)
    out = _emb_lookup(idx, table)
    return out.reshape(BATCH, HIST, D)


# speedup vs baseline: 1.0172x; 1.0172x over previous
import functools

import jax
import jax.numpy as jnp
from jax import lax
from jax.experimental import pallas as pl
from jax.experimental.pallas import tpu as pltpu
from jax.experimental.pallas import tpu_sc as plsc

BATCH = 4096
HIST = 200
D = 64
B = BATCH * HIST
NC = 2
NS = 16
NW = NC * NS
R = B // NW
CH = 128
NCH = R // CH
K = 5
NG = NCH // K

_mesh = plsc.VectorSubcoreMesh(core_axis_name="c", subcore_axis_name="s")


@functools.partial(
    pl.kernel,
    mesh=_mesh,
    out_type=jax.ShapeDtypeStruct((NW, NG, K, CH, D), jnp.float32),
    scratch_types=[
        pltpu.VMEM((NCH, CH), jnp.int32),
        pltpu.VMEM((2, K, CH, D), jnp.float32),
        pltpu.SemaphoreType.DMA,
        pltpu.SemaphoreType.DMA,
        pltpu.SemaphoreType.DMA,
        pltpu.SemaphoreType.DMA,
    ],
    compiler_params=pltpu.CompilerParams(use_tc_tiling_on_sc=False),
)
def _emb_lookup(idx_hbm, table_hbm, out_hbm, idx_v, rows_v, gsem0, gsem1,
                wsem0, wsem1):
    cid = lax.axis_index("c")
    sid = lax.axis_index("s")
    wid = sid * NC + cid
    pltpu.sync_copy(idx_hbm.at[wid], idx_v)

    def fire_gathers(g, b, sem):
        for j in range(K):
            pltpu.async_copy(
                table_hbm.at[idx_v.at[g * K + j]], rows_v.at[b, j], sem
            )

    def drain_gathers(b, sem):
        for j in range(K):
            pltpu.make_async_copy(
                table_hbm.at[idx_v.at[0]], rows_v.at[b, j], sem
            ).wait()

    def wait_write(b, sem):
        pltpu.make_async_copy(rows_v.at[b], out_hbm.at[wid, 0], sem).wait()

    fire_gathers(0, 0, gsem0)

    def pair(p, _):
        @pl.when(p >= 1)
        def _():
            wait_write(1, wsem1)
        fire_gathers(2 * p + 1, 1, gsem1)
        drain_gathers(0, gsem0)
        pltpu.async_copy(rows_v.at[0], out_hbm.at[wid, 2 * p], wsem0)

        @pl.when(p < NG // 2 - 1)
        def _():
            wait_write(0, wsem0)
            fire_gathers(2 * p + 2, 0, gsem0)
        drain_gathers(1, gsem1)
        pltpu.async_copy(rows_v.at[1], out_hbm.at[wid, 2 * p + 1], wsem1)
        return 0

    lax.fori_loop(0, NG // 2, pair, 0)
    wait_write(0, wsem0)
    wait_write(1, wsem1)


def kernel(items, table):
    idx = items.astype(jnp.int32).reshape(NW, NCH, CH)
    out = _emb_lookup(idx, table)
    return out.reshape(BATCH, HIST, D)


# speedup vs baseline: 1.0180x; 1.0007x over previous
import functools

import jax
import jax.numpy as jnp
from jax import lax
from jax.experimental import pallas as pl
from jax.experimental.pallas import tpu as pltpu
from jax.experimental.pallas import tpu_sc as plsc

BATCH = 4096
HIST = 200
D = 64
NC = 2
NS = 16
NW = NC * NS
RB = BATCH // NW
G = 2
NG = RB // G
CHUNKS = ((0, 128), (128, 72))

_mesh = plsc.VectorSubcoreMesh(core_axis_name="c", subcore_axis_name="s")


@functools.partial(
    pl.kernel,
    mesh=_mesh,
    out_type=jax.ShapeDtypeStruct((BATCH, HIST, D), jnp.float32),
    scratch_types=[
        pltpu.VMEM((RB, HIST), jnp.int32),
        pltpu.VMEM((2, G, HIST, D), jnp.float32),
        pltpu.SemaphoreType.DMA,
        pltpu.SemaphoreType.DMA,
        pltpu.SemaphoreType.DMA,
        pltpu.SemaphoreType.DMA,
    ],
    compiler_params=pltpu.CompilerParams(use_tc_tiling_on_sc=False),
)
def _emb_lookup(idx_hbm, table_hbm, out_hbm, idx_v, rows_v, gsem0, gsem1,
                wsem0, wsem1):
    cid = lax.axis_index("c")
    sid = lax.axis_index("s")
    wid = sid * NC + cid
    row0 = wid * RB
    pltpu.sync_copy(idx_hbm.at[pl.ds(row0, RB)], idx_v)

    def fire_gathers(g, b, sem):
        for i in range(G):
            for off, n in CHUNKS:
                pltpu.async_copy(
                    table_hbm.at[idx_v.at[g * G + i, pl.ds(off, n)]],
                    rows_v.at[b, i, pl.ds(off, n)],
                    sem,
                )

    def drain_gathers(b, sem):
        for i in range(G):
            for off, n in CHUNKS:
                pltpu.make_async_copy(
                    table_hbm.at[idx_v.at[0, pl.ds(0, n)]],
                    rows_v.at[b, i, pl.ds(off, n)],
                    sem,
                ).wait()

    def wait_write(b, sem):
        pltpu.make_async_copy(
            rows_v.at[b], out_hbm.at[pl.ds(0, G)], sem
        ).wait()

    fire_gathers(0, 0, gsem0)

    def pair(p, _):
        @pl.when(p >= 1)
        def _():
            wait_write(1, wsem1)
        fire_gathers(2 * p + 1, 1, gsem1)
        drain_gathers(0, gsem0)
        pltpu.async_copy(
            rows_v.at[0], out_hbm.at[pl.ds(row0 + 2 * p * G, G)], wsem0
        )

        @pl.when(p < NG // 2 - 1)
        def _():
            wait_write(0, wsem0)
            fire_gathers(2 * p + 2, 0, gsem0)
        drain_gathers(1, gsem1)
        pltpu.async_copy(
            rows_v.at[1], out_hbm.at[pl.ds(row0 + (2 * p + 1) * G, G)], wsem1
        )
        return 0

    lax.fori_loop(0, NG // 2, pair, 0)
    wait_write(0, wsem0)
    wait_write(1, wsem1)


def kernel(items, table):
    return _emb_lookup(items.astype(jnp.int32), table)
